# Initial kernel scaffold; baseline (speedup 1.0000x reference)
import jax
import jax.numpy as jnp
from jax.experimental import pallas as pl


def kernel(x, segmap, style_codes, exist_codes, fc_W, fc_b, conv_gamma_W, conv_gamma_b, conv_beta_W, conv_beta_b, bn_weight, bn_bias):
    raise NotImplementedError("write your pallas kernel here")


# speedup vs baseline: 6.4843x; 6.4843x over previous
import jax
import jax.numpy as jnp
from jax.experimental import pallas as pl
from jax.experimental.pallas import tpu as pltpu

_B, _C, _H, _W = 2, 96, 224, 224
_S = 256
_NC = 8
_ROWS = 8
_NH = _H // _ROWS


def _stats_label_body(x_ref, seg_ref, sums_ref, lab_ref, acc_ref, accsq_ref):
    b = pl.program_id(0)
    h = pl.program_id(1)
    xb = x_ref[0]

    @pl.when(jnp.logical_and(b == 0, h == 0))
    def _init():
        acc_ref[...] = jnp.zeros_like(acc_ref)
        accsq_ref[...] = jnp.zeros_like(accsq_ref)

    acc_ref[...] += xb
    accsq_ref[...] += xb * xb

    seg = seg_ref[0]
    cls = jax.lax.broadcasted_iota(jnp.int32, (_NC, _ROWS, _W), 0).astype(jnp.float32)
    cand = jnp.where(seg != 0.0, cls, -1.0)
    lab_ref[0] = jnp.max(cand, axis=0)

    @pl.when(jnp.logical_and(b == _B - 1, h == _NH - 1))
    def _finish():
        s1 = jnp.sum(acc_ref[...], axis=(1, 2))
        s2 = jnp.sum(accsq_ref[...], axis=(1, 2))
        sums_ref[...] = jnp.stack([s1, s2], axis=0)


def _table_body(style_ref, exist_ref, fcWT_ref, fcb_ref, wflat_ref, g16_ref):
    mus = []
    for j in range(_NC):
        ex = exist_ref[:, j : j + 1]
        code = jnp.where(ex == 1.0, style_ref[:, j, :], style_ref[:, _NC, :])
        mu = jnp.maximum(
            jnp.dot(code, fcWT_ref[j], preferred_element_type=jnp.float32)
            + fcb_ref[j][None, :],
            0.0,
        )
        mus.append(mu)
    MU = jnp.concatenate(mus, axis=0)
    g16_ref[...] = jnp.dot(MU, wflat_ref[...], preferred_element_type=jnp.float32)


def _apply_body(x_ref, lab_ref, g_ref, chan_ref, out_ref):
    h = pl.program_id(1)
    r0 = h * _ROWS
    g = g_ref[0]
    scale = chan_ref[:, 0:1]
    off = chan_ref[:, 1:2]
    g1 = chan_ref[:, 2:3]
    bb = chan_ref[:, 3:4]
    for r in range(_ROWS):
        rows = []
        for dy in range(3):
            lrow = lab_ref[0, r0 + r + dy, :]
            for dx in range(3):
                seg = jax.lax.slice(lrow, (dx,), (dx + _W,))
                cmp = seg[None, :] == jax.lax.broadcasted_iota(
                    jnp.int32, (_NC, _W), 0
                ).astype(jnp.float32)
                rows.append(cmp.astype(jnp.float32))
        a = jnp.concatenate(rows, axis=0)
        mm = jnp.dot(g, a, preferred_element_type=jnp.float32)
        xr = x_ref[0, :, r, :]
        n = xr * scale + off
        out_ref[0, :, r, :] = n * (g1 + mm[:_C, :]) + mm[_C:, :] + bb


def kernel(x, segmap, style_codes, exist_codes, fc_W, fc_b, conv_gamma_W,
           conv_gamma_b, conv_beta_W, conv_beta_b, bn_weight, bn_bias):
    f32 = jnp.float32

    sums, lab = pl.pallas_call(
        _stats_label_body,
        grid=(_B, _NH),
        in_specs=[
            pl.BlockSpec((1, _C, _ROWS, _W), lambda b, h: (b, 0, h, 0)),
            pl.BlockSpec((1, _NC, _ROWS, _W), lambda b, h: (b, 0, h, 0)),
        ],
        out_specs=[
            pl.BlockSpec((2, _C), lambda b, h: (0, 0)),
            pl.BlockSpec((1, _ROWS, _W), lambda b, h: (b, h, 0)),
        ],
        out_shape=[
            jax.ShapeDtypeStruct((2, _C), f32),
            jax.ShapeDtypeStruct((_B, _H, _W), f32),
        ],
        scratch_shapes=[
            pltpu.VMEM((_C, _ROWS, _W), f32),
            pltpu.VMEM((_C, _ROWS, _W), f32),
        ],
    )(x, segmap)

    wg = jnp.transpose(conv_gamma_W, (1, 0, 2, 3)).reshape(_S, _C * 9)
    wb = jnp.transpose(conv_beta_W, (1, 0, 2, 3)).reshape(_S, _C * 9)
    wflat = jnp.concatenate([wg, wb], axis=1)
    fcWT = jnp.transpose(fc_W, (0, 2, 1))
    g16 = pl.pallas_call(
        _table_body,
        out_shape=jax.ShapeDtypeStruct((_NC * _B, 2 * _C * 9), f32),
    )(style_codes, exist_codes.astype(f32), fcWT, fc_b, wflat)
    g = (
        g16.reshape(_NC, _B, 2 * _C, 9)
        .transpose(1, 2, 3, 0)
        .reshape(_B, 2 * _C, 9 * _NC)
    )

    n_pix = _B * _H * _W
    mean = sums[0] / n_pix
    var = sums[1] / n_pix - mean * mean
    scale = bn_weight * jax.lax.rsqrt(var + 1e-5)
    off = bn_bias - mean * scale
    chan = jnp.stack([scale, off, 1.0 + conv_gamma_b, conv_beta_b], axis=1)

    labpad = jnp.pad(lab, ((0, 0), (1, 1), (1, 1)), constant_values=-1.0)

    out = pl.pallas_call(
        _apply_body,
        grid=(_B, _NH),
        in_specs=[
            pl.BlockSpec((1, _C, _ROWS, _W), lambda b, h: (b, 0, h, 0)),
            pl.BlockSpec((1, _H + 2, _W + 2), lambda b, h: (b, 0, 0)),
            pl.BlockSpec((1, 2 * _C, 9 * _NC), lambda b, h: (b, 0, 0)),
            pl.BlockSpec((_C, 4), lambda b, h: (0, 0)),
        ],
        out_specs=pl.BlockSpec((1, _C, _ROWS, _W), lambda b, h: (b, 0, h, 0)),
        out_shape=jax.ShapeDtypeStruct((_B, _C, _H, _W), f32),
    )(x, labpad, g, chan)
    return out
